# baseline (speedup 1.0000x reference)
import functools

import jax
import jax.numpy as jnp
from jax import lax
from jax.experimental import pallas as pl
from jax.experimental.pallas import tpu as pltpu
from jax.experimental.pallas import tpu_sc as plsc

_CHUNK = 128


@functools.lru_cache(maxsize=None)
def _build_gather(B: int, Wt: int):
    info = plsc.get_sparse_core_info()
    NC, NS, L = info.num_cores, info.num_subcores, info.num_lanes
    NW = NC * NS
    bpw = B // NW
    nch = bpw // _CHUNK
    assert B % (NW * L) == 0 and bpw % _CHUNK == 0
    row_tile_words = 8 * Wt
    col_tile_words = 8 * 128
    mesh = plsc.VectorSubcoreMesh(core_axis_name="c", subcore_axis_name="s")

    @functools.partial(
        pl.kernel,
        mesh=mesh,
        out_type=jax.ShapeDtypeStruct((B,), jnp.float32),
        scratch_types=[
            pltpu.VMEM((2, bpw), jnp.int32),
            pltpu.VMEM((nch, _CHUNK), jnp.int32),
            pltpu.VMEM((bpw,), jnp.float32),
            pltpu.SemaphoreType.DMA,
        ],
    )
    def body(obs_hbm, table_hbm, out_hbm, pairs_v, idx_v, val_v, sem):
        wid = lax.axis_index("s") * NC + lax.axis_index("c")
        base = wid * bpw
        pltpu.sync_copy(obs_hbm.at[:, pl.ds(base, bpw)], pairs_v)
        per_chunk = _CHUNK // L
        copies = []
        for j in range(nch):
            for k in range(per_chunk):
                i = j * per_chunk + k
                r = pairs_v[0, pl.ds(i * L, L)]
                c = pairs_v[1, pl.ds(i * L, L)]
                phys = (
                    lax.shift_right_logical(r, 3) * row_tile_words
                    + lax.shift_right_logical(c, 7) * col_tile_words
                    + lax.shift_left(lax.bitwise_and(r, jnp.int32(7)), 7)
                    + lax.bitwise_and(c, jnp.int32(127))
                )
                idx_v[j, pl.ds(k * L, L)] = phys
            copies.append(pltpu.async_copy(
                table_hbm.at[idx_v.at[j]],
                val_v.at[pl.ds(j * _CHUNK, _CHUNK)],
                sem,
            ))
        for cp in copies:
            cp.wait()
        pltpu.sync_copy(val_v, out_hbm.at[pl.ds(base, bpw)])

    return body


def kernel(obs, acs, cost_matrix):
    del acs
    B = obs.shape[0]
    H, W = cost_matrix.shape
    obs_t = obs.astype(jnp.int32).T
    wp = -W % 128
    hp = -H % 8
    padded = jnp.pad(cost_matrix, ((0, hp), (0, wp)))
    Ht, Wt = H + hp, W + wp
    phys = (
        padded.reshape(Ht // 8, 8, Wt // 128, 128)
        .transpose(0, 2, 1, 3)
        .reshape(Ht * Wt)
    )
    return _build_gather(B, Wt)(obs_t, phys)
